# baseline (speedup 1.0000x reference)
import functools

import jax
import jax.numpy as jnp
from jax import lax
from jax.experimental import pallas as pl
from jax.experimental.pallas import tpu as pltpu
from jax.experimental.pallas import tpu_sc as plsc

N = 10000
E = 320000
DE = 16
G = 64
NT = 32
EPT = E // NT
EB = 80
NB = EPT // EB
RB = 1000
NRB = N // RB
F = 128

_DIMS = [(128, 512), (512, 256), (256, 128), (128, 256), (256, 512)]


def _ke_body(ea_ref, w0, w1, w2, w3, w4, aep_ref, out_ref, sum_ref):
    blk = ea_ref[...]
    ws = (w0, w1, w2, w3, w4)
    cols = []
    for l in range(5):
        dout = _DIMS[l][1]
        e = jnp.dot(blk, ws[l][...], preferred_element_type=jnp.float32)
        al = jnp.sum(e * aep_ref[l:l + 1, :dout], axis=1)
        cols.append(al[:, None])
    cols.append(jnp.zeros((blk.shape[0], 3), jnp.float32))
    out_ref[...] = jnp.concatenate(cols, axis=1)

    @pl.when(pl.program_id(0) == 0)
    def _():
        sum_ref[...] = jnp.zeros((8, DE), jnp.float32)

    s = jnp.sum(blk, axis=0)
    sum_ref[...] += jnp.concatenate(
        [s[None, :], jnp.zeros((7, DE), jnp.float32)], axis=0)


def _alpha_e(edge_attr, wes, aep):
    ekb = 2000
    grid = (E // ekb,)
    outs = [jax.ShapeDtypeStruct((E, 8), jnp.float32),
            jax.ShapeDtypeStruct((8, DE), jnp.float32)]
    res = pl.pallas_call(
        _ke_body,
        grid=grid,
        in_specs=[pl.BlockSpec((ekb, DE), lambda i: (i, 0))]
        + [pl.BlockSpec((DE, _DIMS[l][1]), lambda i: (0, 0))
           for l in range(5)]
        + [pl.BlockSpec((8, 512), lambda i: (0, 0))],
        out_specs=[pl.BlockSpec((ekb, 8), lambda i: (i, 0)),
                   pl.BlockSpec((8, DE), lambda i: (0, 0))],
        out_shape=outs,
    )(edge_attr, *wes, aep)
    ae8, sum_ea = res
    return [ae8[:, l] for l in range(5)], sum_ea


def _mm_body(ncin, nc, norm, *refs):
    if norm:
        zrefs = refs[:ncin]
        strefs = refs[ncin:2 * ncin]
        g_ref, be_ref, w_ref, aa_ref = refs[2 * ncin:2 * ncin + 4]
        orefs = refs[2 * ncin + 4:]
    else:
        zrefs = refs[:ncin]
        w_ref, aa_ref = refs[ncin:ncin + 2]
        orefs = refs[ncin + 2:]
    hrefs = orefs[:nc]
    sd_ref = orefs[nc]

    acc = None
    for cin in range(ncin):
        z = zrefs[cin][...]
        if norm:
            st = strefs[cin][...]
            m = st[0:1, :] / N
            var = st[1:2, :] / N - m * m
            inv = lax.rsqrt(var + 1e-5)
            z = (z - m) * inv * g_ref[cin:cin + 1, :] + be_ref[cin:cin + 1, :]
        h = jnp.dot(z, w_ref[cin], preferred_element_type=jnp.float32)
        acc = h if acc is None else acc + h
    for c in range(nc):
        hrefs[c][...] = acc[:, c * F:(c + 1) * F]
    s = jnp.sum(acc * aa_ref[0:1, :], axis=1)
    dv = jnp.sum(acc * aa_ref[1:2, :], axis=1)
    sd_ref[...] = jnp.concatenate(
        [s[:, None], dv[:, None],
         jnp.zeros((acc.shape[0], F - 2), jnp.float32)], axis=1)


def _matmul_layer(zchunks, stats, g8, be8, w3, aa, nc):
    ncin = len(zchunks)
    norm = stats is not None
    dout = nc * F
    blk = pl.BlockSpec((RB, F), lambda i: (i, 0))
    in_specs = [blk] * ncin
    args = list(zchunks)
    if norm:
        in_specs += [pl.BlockSpec((8, F), lambda i: (0, 0))] * ncin
        args += list(stats)
        in_specs += [pl.BlockSpec((8, F), lambda i: (0, 0))] * 2
        args += [g8, be8]
    in_specs += [
        pl.BlockSpec((ncin, F, dout), lambda i: (0, 0, 0)),
        pl.BlockSpec((8, dout), lambda i: (0, 0)),
    ]
    args += [w3, aa]
    out_shape = [jax.ShapeDtypeStruct((N, F), jnp.float32)] * (nc + 1)
    res = pl.pallas_call(
        functools.partial(_mm_body, ncin, nc, norm),
        grid=(NRB,),
        in_specs=in_specs,
        out_specs=[blk] * (nc + 1),
        out_shape=out_shape,
    )(*args)
    return list(res[:nc]), res[nc]


def _mesh():
    return plsc.VectorSubcoreMesh(
        core_axis_name="c", subcore_axis_name="s",
        num_cores=2, num_subcores=16)


def _sc_alpha_body(src3, dst3, ae3, s_h, d_h, z1d,
                   ex_o, den0_o, den1_o,
                   s_v, d_v, ae_v, src_v, dst_v, ex_v, zd_v, td_v,
                   den_s, sem):
    cid = lax.axis_index("c")
    sid = lax.axis_index("s")
    wid = cid * 16 + sid

    pltpu.sync_copy(s_h, s_v)
    pltpu.sync_copy(d_h, d_v)
    pltpu.sync_copy(ae3.at[wid], ae_v)
    pltpu.sync_copy(src3.at[wid], src_v)
    pltpu.sync_copy(dst3.at[wid], dst_v)

    @pl.when(sid < 10)
    def _():
        pltpu.sync_copy(z1d.at[pl.ds(sid * 1000, 1000)], zd_v)
        pltpu.sync_copy(zd_v, den_s.at[pl.ds(sid * 1000, 1000)])

    @pl.loop(0, NB)
    def _(b):
        for j in range(5):
            o = j * 16
            srcv = src_v[b, pl.ds(o, 16)]
            dstv = dst_v[b, pl.ds(o, 16)]
            sv = plsc.load_gather(s_v, [srcv])
            dv = plsc.load_gather(d_v, [dstv])
            a = sv + dv + ae_v[b, pl.ds(o, 16)]
            a = jnp.maximum(a, a * 0.2)
            ex_v[b, pl.ds(o, 16)] = jnp.exp(a)

    pltpu.sync_copy(ex_v, ex_o.at[wid])
    plsc.subcore_barrier()

    @pl.loop(0, NB)
    def _(b):
        pltpu.sync_copy(ex_v.at[b], den_s.at[dst_v.at[b]], add=True)

    plsc.subcore_barrier()

    @pl.when(sid < 10)
    def _():
        pltpu.sync_copy(den_s.at[pl.ds(sid * 1000, 1000)], td_v)

        @pl.when(cid == 0)
        def _():
            pltpu.sync_copy(td_v, den0_o.at[pl.ds(sid * 1000, 1000)])

        @pl.when(cid == 1)
        def _():
            pltpu.sync_copy(td_v, den1_o.at[pl.ds(sid * 1000, 1000)])


def _sc_agg_body(nc, *refs):
    (src3, dst3, ex4, z128) = refs[:4]
    hcs = refs[4:4 + nc]
    parts_o = refs[4 + nc:4 + 2 * nc]
    (src_v, dst_v, exb_v, rows_v, tmp_v, acc_s, sem) = refs[4 + 2 * nc:]

    cid = lax.axis_index("c")
    sid = lax.axis_index("s")
    wid = cid * 16 + sid

    pltpu.sync_copy(src3.at[wid], src_v)
    pltpu.sync_copy(dst3.at[wid], dst_v)

    for c in range(nc):
        @pl.when(sid < 10)
        def _():
            pltpu.sync_copy(z128.at[pl.ds(0, 40)], tmp_v)

            @pl.loop(0, 25)
            def _(k):
                pltpu.sync_copy(
                    tmp_v, acc_s.at[pl.ds(sid * 1000 + k * 40, 40)])

        plsc.subcore_barrier()

        @pl.loop(0, NB)
        def _(b):
            pltpu.sync_copy(ex4.at[wid, b], exb_v)
            pltpu.async_copy(hcs[c].at[src_v.at[b]], rows_v, sem).wait()

            @pl.loop(0, EB // 16)
            def _(g):
                exv = exb_v[0, pl.ds(g * 16, 16)]
                for rr in range(16):
                    e = exv[rr]
                    r = g * 16 + rr
                    for j in range(8):
                        rows_v[r, pl.ds(j * 16, 16)] = (
                            rows_v[r, pl.ds(j * 16, 16)] * e)

            pltpu.sync_copy(rows_v, acc_s.at[dst_v.at[b]], add=True)

        plsc.subcore_barrier()

        @pl.when(sid < 10)
        def _(c=c):
            @pl.loop(0, 25)
            def _(k):
                pltpu.sync_copy(
                    acc_s.at[pl.ds(sid * 1000 + k * 40, 40)], tmp_v)
                pltpu.sync_copy(
                    tmp_v,
                    parts_o[c].at[cid, pl.ds(sid * 1000 + k * 40, 40)])


def _sc_layer(nc, src3, dst3, ae3, s, d, z128, z1d, hcs):
    alpha = pl.kernel(
        _sc_alpha_body,
        out_type=[jax.ShapeDtypeStruct((NT, NB, EB), jnp.float32),
                  jax.ShapeDtypeStruct((N,), jnp.float32),
                  jax.ShapeDtypeStruct((N,), jnp.float32)],
        mesh=_mesh(),
        scratch_types=[
            pltpu.VMEM((N,), jnp.float32),
            pltpu.VMEM((N,), jnp.float32),
            pltpu.VMEM((NB, EB), jnp.float32),
            pltpu.VMEM((NB, EB), jnp.int32),
            pltpu.VMEM((NB, EB), jnp.int32),
            pltpu.VMEM((NB, EB), jnp.float32),
            pltpu.VMEM((1000,), jnp.float32),
            pltpu.VMEM((1000,), jnp.float32),
            pltpu.VMEM_SHARED((N,), jnp.float32),
            pltpu.SemaphoreType.DMA,
        ],
        compiler_params=pltpu.CompilerParams(needs_layout_passes=False),
    )
    ex3, den0, den1 = alpha(src3, dst3, ae3, s, d, z1d)

    agg = pl.kernel(
        functools.partial(_sc_agg_body, nc),
        out_type=[jax.ShapeDtypeStruct((2, N, F), jnp.float32)] * nc,
        mesh=_mesh(),
        scratch_types=[
            pltpu.VMEM((NB, EB), jnp.int32),
            pltpu.VMEM((NB, EB), jnp.int32),
            pltpu.VMEM((1, EB), jnp.float32),
            pltpu.VMEM((EB, F), jnp.float32),
            pltpu.VMEM((40, F), jnp.float32),
            pltpu.VMEM_SHARED((N, F), jnp.float32),
            pltpu.SemaphoreType.DMA,
        ],
        compiler_params=pltpu.CompilerParams(needs_layout_passes=False),
    )
    ex4 = ex3.reshape(NT, NB, 1, EB)
    parts = agg(src3, dst3, ex4, z128, *hcs)
    return jnp.stack([den0, den1]), list(parts)


def _post_body(nc, *refs):
    parts = refs[:nc]
    hrefs = refs[nc:2 * nc]
    sd_ref = refs[2 * nc]
    den_ref = refs[2 * nc + 1]
    cl_ref = refs[2 * nc + 2]
    bp_ref = refs[2 * nc + 3]
    yrefs = refs[2 * nc + 4:3 * nc + 4]
    strefs = refs[3 * nc + 4:]

    sd = sd_ref[...]
    cl = cl_ref[0:1, 0:1]
    al = sd[:, 0:1] + sd[:, 1:2] + cl
    al = jnp.maximum(al, al * 0.2)
    exl = jnp.exp(al)
    den = den_ref[...]
    dtot = den[0] + den[1] + exl + 1e-16

    i = pl.program_id(0)
    for c in range(nc):
        p = parts[c][...]
        h = hrefs[c][...]
        y = (p[0] + p[1] + exl * h) / dtot + bp_ref[c:c + 1, :]
        y = jnp.maximum(y, y * 0.01)
        yrefs[c][...] = y

        @pl.when(i == 0)
        def _():
            strefs[c][...] = jnp.zeros((8, F), jnp.float32)

        srow = jnp.sum(y, axis=0)
        sqrow = jnp.sum(y * y, axis=0)
        strefs[c][...] += jnp.concatenate(
            [srow[None, :], sqrow[None, :], jnp.zeros((6, F), jnp.float32)],
            axis=0)


def _post_layer(nc, parts, hcs, sdpack, denp3, clpack, bp8):
    blk = pl.BlockSpec((RB, F), lambda i: (i, 0))
    cblk = pl.BlockSpec((8, F), lambda i: (0, 0))
    in_specs = [pl.BlockSpec((2, RB, F), lambda i: (0, i, 0))] * nc
    in_specs += [blk] * nc
    in_specs += [blk,
                 pl.BlockSpec((2, RB, 1), lambda i: (0, i, 0)),
                 cblk, cblk]
    out_shape = ([jax.ShapeDtypeStruct((N, F), jnp.float32)] * nc
                 + [jax.ShapeDtypeStruct((8, F), jnp.float32)] * nc)
    res = pl.pallas_call(
        functools.partial(_post_body, nc),
        grid=(NRB,),
        in_specs=in_specs,
        out_specs=[blk] * nc + [cblk] * nc,
        out_shape=out_shape,
    )(*parts, *hcs, sdpack, denp3, clpack, bp8)
    return list(res[:nc]), list(res[nc:])


def _pool_body(ncin, *refs):
    zrefs = refs[:ncin]
    strefs = refs[ncin:2 * ncin]
    (g_ref, be_ref, b3_ref, fw0_ref, fw1_ref, fw2_ref, fb_ref,
     pool_ref, out_ref) = refs[2 * ncin:]

    i = pl.program_id(0)
    zs = []
    for cin in range(ncin):
        z = zrefs[cin][...]
        st = strefs[cin][...]
        m = st[0:1, :] / N
        var = st[1:2, :] / N - m * m
        inv = lax.rsqrt(var + 1e-5)
        zs.append((z - m) * inv * g_ref[cin:cin + 1, :]
                  + be_ref[cin:cin + 1, :])
    zfull = jnp.concatenate(zs, axis=1)
    b = b3_ref[0, 0, :]
    oh = (lax.broadcasted_iota(jnp.int32, (G, RB), 0)
          == b[None, :]).astype(jnp.float32)

    @pl.when(i == 0)
    def _():
        pool_ref[...] = jnp.zeros_like(pool_ref)

    pool_ref[...] += jnp.dot(oh, zfull, preferred_element_type=jnp.float32,
                             precision=lax.Precision.HIGHEST)

    @pl.when(i == NRB - 1)
    def _():
        p = pool_ref[...]
        z1 = jnp.dot(p, fw0_ref[...],
                     preferred_element_type=jnp.float32) + fb_ref[0:1, :]
        z1 = jnp.maximum(z1, 0.0)
        z2 = jnp.dot(z1, fw1_ref[...],
                     preferred_element_type=jnp.float32) + fb_ref[1:2, :16]
        z2 = jnp.maximum(z2, 0.0)
        z3 = jnp.dot(z2, fw2_ref[...],
                     preferred_element_type=jnp.float32) + fb_ref[2:3, :8]
        out_ref[...] = z3


def _pool_layer(ncin, zchunks, stats, g8, be8, batch3, fw0, fw1, fw2p, fbp):
    blk = pl.BlockSpec((RB, F), lambda i: (i, 0))
    cblk = pl.BlockSpec((8, F), lambda i: (0, 0))
    din = ncin * F
    in_specs = [blk] * ncin + [cblk] * ncin
    in_specs += [cblk, cblk,
                 pl.BlockSpec((1, 1, RB), lambda i: (i, 0, 0)),
                 pl.BlockSpec((din, F), lambda i: (0, 0)),
                 pl.BlockSpec((F, 16), lambda i: (0, 0)),
                 pl.BlockSpec((16, 8), lambda i: (0, 0))]
    in_specs += [cblk]
    out_shape = [jax.ShapeDtypeStruct((G, din), jnp.float32),
                 jax.ShapeDtypeStruct((G, 8), jnp.float32)]
    res = pl.pallas_call(
        functools.partial(_pool_body, ncin),
        grid=(NRB,),
        in_specs=in_specs,
        out_specs=[pl.BlockSpec((G, din), lambda i: (0, 0)),
                   pl.BlockSpec((G, 8), lambda i: (0, 0))],
        out_shape=out_shape,
    )(*zchunks, *stats, g8, be8, batch3, fw0, fw1, fw2p, fbp)
    return res[1]


def _pad8(a):
    rows, cols = a.shape
    return jnp.concatenate(
        [a, jnp.zeros((8 - rows, cols), jnp.float32)], axis=0)


def kernel(x, edge_index, edge_attr, batch, params):
    wes = [params['We%d' % l] for l in range(5)]
    aep = jnp.zeros((8, 512), jnp.float32)
    for l in range(5):
        aep = aep.at[l, :_DIMS[l][1]].set(params['ae%d' % l])
    w3s, aas, g8s, be8s, bp8s = [], [], [], [], []
    for l, (din, dout) in enumerate(_DIMS):
        W = params['W%d' % l]
        aa = jnp.zeros((8, dout), jnp.float32)
        aa = aa.at[0, :].set(params['as%d' % l])
        aa = aa.at[1, :].set(params['ad%d' % l])
        aas.append(aa)
        w3s.append(W.reshape(din // F, F, dout))
        g8s.append(_pad8(params['g%d' % l].reshape(dout // F, F)))
        be8s.append(_pad8(params['beta%d' % l].reshape(dout // F, F)))
        bp8s.append(_pad8(params['b%d' % l].reshape(dout // F, F)))
    fw0 = params['fcw0']
    fw1 = params['fcw1']
    fw2p = jnp.concatenate(
        [params['fcw2'], jnp.zeros((16, 7), jnp.float32)], axis=1)
    fbp = jnp.zeros((8, F), jnp.float32)
    fbp = fbp.at[0, :].set(params['fcb0'])
    fbp = fbp.at[1, :16].set(params['fcb1'])
    fbp = fbp.at[2, :1].set(params['fcb2'])

    ae_list, sum_ea = _alpha_e(edge_attr, wes, aep)
    mean_ea = sum_ea[0, :] / E
    cvals = jnp.stack([(mean_ea @ wes[l]) @ params['ae%d' % l]
                       for l in range(5)])

    src3 = edge_index[0].reshape(NT, NB, EB)
    dst3 = edge_index[1].reshape(NT, NB, EB)
    z128 = jnp.zeros((N, F), jnp.float32)
    z1d = jnp.zeros((N,), jnp.float32)
    batch3 = batch.reshape(NRB, 1, RB)

    zchunks = [x]
    stats = None
    for l, (din, dout) in enumerate(_DIMS):
        nc = dout // F
        hcs, sdpack = _matmul_layer(
            zchunks, stats, None if stats is None else g8s[l - 1],
            None if stats is None else be8s[l - 1],
            w3s[l], aas[l], nc)
        s = sdpack[:, 0]
        d = sdpack[:, 1]
        ae3 = ae_list[l].reshape(NT, NB, EB)
        denomp, parts = _sc_layer(
            nc, src3, dst3, ae3, s, d, z128, z1d, hcs)
        denp3 = denomp.reshape(2, N, 1)
        clpack = jnp.full((8, F), cvals[l], jnp.float32)
        zchunks, stats = _post_layer(
            nc, parts, hcs, sdpack, denp3, clpack, bp8s[l])

    out8 = _pool_layer(
        len(zchunks), zchunks, stats, g8s[4], be8s[4], batch3,
        fw0, fw1, fw2p, fbp)
    return out8[:, 0]


# speedup vs baseline: 1.6729x; 1.6729x over previous
import functools

import jax
import jax.numpy as jnp
from jax import lax
from jax.experimental import pallas as pl
from jax.experimental.pallas import tpu as pltpu
from jax.experimental.pallas import tpu_sc as plsc

N = 10000
E = 320000
DE = 16
G = 64
NT = 32
EPT = E // NT
EB = 80
NB = EPT // EB
RB = 1000
NRB = N // RB
F = 128

_DIMS = [(128, 512), (512, 256), (256, 128), (128, 256), (256, 512)]


def _ke_body(ea_ref, w0, w1, w2, w3, w4, aep_ref, out_ref, sum_ref):
    blk = ea_ref[...]
    ws = (w0, w1, w2, w3, w4)
    cols = []
    for l in range(5):
        dout = _DIMS[l][1]
        e = jnp.dot(blk, ws[l][...], preferred_element_type=jnp.float32)
        al = jnp.sum(e * aep_ref[l:l + 1, :dout], axis=1)
        cols.append(al[:, None])
    cols.append(jnp.zeros((blk.shape[0], 3), jnp.float32))
    out_ref[...] = jnp.concatenate(cols, axis=1)

    @pl.when(pl.program_id(0) == 0)
    def _():
        sum_ref[...] = jnp.zeros((8, DE), jnp.float32)

    s = jnp.sum(blk, axis=0)
    sum_ref[...] += jnp.concatenate(
        [s[None, :], jnp.zeros((7, DE), jnp.float32)], axis=0)


def _alpha_e(edge_attr, wes, aep):
    ekb = 2000
    grid = (E // ekb,)
    outs = [jax.ShapeDtypeStruct((E, 8), jnp.float32),
            jax.ShapeDtypeStruct((8, DE), jnp.float32)]
    res = pl.pallas_call(
        _ke_body,
        grid=grid,
        in_specs=[pl.BlockSpec((ekb, DE), lambda i: (i, 0))]
        + [pl.BlockSpec((DE, _DIMS[l][1]), lambda i: (0, 0))
           for l in range(5)]
        + [pl.BlockSpec((8, 512), lambda i: (0, 0))],
        out_specs=[pl.BlockSpec((ekb, 8), lambda i: (i, 0)),
                   pl.BlockSpec((8, DE), lambda i: (0, 0))],
        out_shape=outs,
    )(edge_attr, *wes, aep)
    ae8, sum_ea = res
    return [ae8[:, l] for l in range(5)], sum_ea


def _mm_body(ncin, nc, norm, *refs):
    if norm:
        zrefs = refs[:ncin]
        strefs = refs[ncin:2 * ncin]
        g_ref, be_ref, w_ref, aa_ref = refs[2 * ncin:2 * ncin + 4]
        orefs = refs[2 * ncin + 4:]
    else:
        zrefs = refs[:ncin]
        w_ref, aa_ref = refs[ncin:ncin + 2]
        orefs = refs[ncin + 2:]
    hrefs = orefs[:nc]
    sd_ref = orefs[nc]

    acc = None
    for cin in range(ncin):
        z = zrefs[cin][...]
        if norm:
            st = strefs[cin][...]
            m = st[0:1, :] / N
            var = st[1:2, :] / N - m * m
            inv = lax.rsqrt(var + 1e-5)
            z = (z - m) * inv * g_ref[cin:cin + 1, :] + be_ref[cin:cin + 1, :]
        h = jnp.dot(z, w_ref[cin], preferred_element_type=jnp.float32)
        acc = h if acc is None else acc + h
    for c in range(nc):
        hrefs[c][...] = acc[:, c * F:(c + 1) * F]
    s = jnp.sum(acc * aa_ref[0:1, :], axis=1)
    dv = jnp.sum(acc * aa_ref[1:2, :], axis=1)
    sd_ref[...] = jnp.concatenate(
        [s[:, None], dv[:, None],
         jnp.zeros((acc.shape[0], F - 2), jnp.float32)], axis=1)


def _matmul_layer(zchunks, stats, g8, be8, w3, aa, nc):
    ncin = len(zchunks)
    norm = stats is not None
    dout = nc * F
    blk = pl.BlockSpec((RB, F), lambda i: (i, 0))
    in_specs = [blk] * ncin
    args = list(zchunks)
    if norm:
        in_specs += [pl.BlockSpec((8, F), lambda i: (0, 0))] * ncin
        args += list(stats)
        in_specs += [pl.BlockSpec((8, F), lambda i: (0, 0))] * 2
        args += [g8, be8]
    in_specs += [
        pl.BlockSpec((ncin, F, dout), lambda i: (0, 0, 0)),
        pl.BlockSpec((8, dout), lambda i: (0, 0)),
    ]
    args += [w3, aa]
    out_shape = [jax.ShapeDtypeStruct((N, F), jnp.float32)] * (nc + 1)
    res = pl.pallas_call(
        functools.partial(_mm_body, ncin, nc, norm),
        grid=(NRB,),
        in_specs=in_specs,
        out_specs=[blk] * (nc + 1),
        out_shape=out_shape,
    )(*args)
    return list(res[:nc]), res[nc]


def _mesh():
    return plsc.VectorSubcoreMesh(
        core_axis_name="c", subcore_axis_name="s",
        num_cores=2, num_subcores=16)


def _sc_alpha_body(src3, dst3, ae3, s_h, d_h, z1d,
                   ex_o, den0_o, den1_o,
                   s_v, d_v, ae_v, src_v, dst_v, ex_v, zd_v, td_v,
                   den_s, sem):
    cid = lax.axis_index("c")
    sid = lax.axis_index("s")
    wid = cid * 16 + sid

    pltpu.sync_copy(s_h, s_v)
    pltpu.sync_copy(d_h, d_v)
    pltpu.sync_copy(ae3.at[wid], ae_v)
    pltpu.sync_copy(src3.at[wid], src_v)
    pltpu.sync_copy(dst3.at[wid], dst_v)

    @pl.when(sid < 10)
    def _():
        pltpu.sync_copy(z1d.at[pl.ds(sid * 1000, 1000)], zd_v)
        pltpu.sync_copy(zd_v, den_s.at[pl.ds(sid * 1000, 1000)])

    @pl.loop(0, NB)
    def _(b):
        for j in range(5):
            o = j * 16
            srcv = src_v[b, pl.ds(o, 16)]
            dstv = dst_v[b, pl.ds(o, 16)]
            sv = plsc.load_gather(s_v, [srcv])
            dv = plsc.load_gather(d_v, [dstv])
            a = sv + dv + ae_v[b, pl.ds(o, 16)]
            a = jnp.maximum(a, a * 0.2)
            ex_v[b, pl.ds(o, 16)] = jnp.exp(a)

    pltpu.sync_copy(ex_v, ex_o.at[wid])
    plsc.subcore_barrier()

    @pl.loop(0, NB)
    def _(b):
        pltpu.sync_copy(ex_v.at[b], den_s.at[dst_v.at[b]], add=True)

    plsc.subcore_barrier()

    @pl.when(sid < 10)
    def _():
        pltpu.sync_copy(den_s.at[pl.ds(sid * 1000, 1000)], td_v)

        @pl.when(cid == 0)
        def _():
            pltpu.sync_copy(td_v, den0_o.at[pl.ds(sid * 1000, 1000)])

        @pl.when(cid == 1)
        def _():
            pltpu.sync_copy(td_v, den1_o.at[pl.ds(sid * 1000, 1000)])


def _sc_agg_body(nc, *refs):
    (pk3, ex4, z128) = refs[:3]
    hcs = refs[3:3 + nc]
    parts_o = refs[3 + nc:3 + 2 * nc]
    (pk_v, srcab, dstab, exab, rows_a, rows_b, tmp_v, acc_s,
     sem) = refs[3 + 2 * nc:]

    cid = lax.axis_index("c")
    sid = lax.axis_index("s")
    wid = cid * 16 + sid

    pltpu.sync_copy(pk3.at[wid], pk_v)

    def unpack(b, par):
        for g in range(EB // 16):
            pk = pk_v[b, pl.ds(g * 16, 16)]
            srcab[par, pl.ds(g * 16, 16)] = lax.bitwise_and(
                pk, jnp.full((16,), 0xFFFF, jnp.int32))
            dstab[par, pl.ds(g * 16, 16)] = lax.shift_right_logical(
                pk, jnp.full((16,), 16, jnp.int32))

    for c in range(nc):
        hc = hcs[c]

        def issue(b, par):
            unpack(b, par)
            pltpu.async_copy(ex4.at[wid, b], exab.at[par], sem)
            pltpu.async_copy(hc.at[srcab.at[par]], _rows(par), sem)

        def _rows(par):
            return rows_a if par == 0 else rows_b

        def wait(b, par):
            pltpu.make_async_copy(ex4.at[wid, b], exab.at[par], sem).wait()
            pltpu.make_async_copy(hc.at[srcab.at[par]], _rows(par),
                                  sem).wait()

        def process(b, par):
            rows = _rows(par)

            @pl.loop(0, EB // 16)
            def _(g):
                exv = exab[par, 0, pl.ds(g * 16, 16)]
                for rr in range(16):
                    e = exv[rr]
                    r = g * 16 + rr
                    for j in range(8):
                        rows[r, pl.ds(j * 16, 16)] = (
                            rows[r, pl.ds(j * 16, 16)] * e)

            pltpu.sync_copy(rows, acc_s.at[dstab.at[par]], add=True)

        @pl.when(sid < 10)
        def _():
            pltpu.sync_copy(z128.at[pl.ds(0, 40)], tmp_v)

            @pl.loop(0, 25)
            def _(k):
                pltpu.sync_copy(
                    tmp_v, acc_s.at[pl.ds(sid * 1000 + k * 40, 40)])

        plsc.subcore_barrier()

        issue(0, 0)

        @pl.loop(0, (NB - 1) // 2)
        def _(k):
            b0 = 2 * k
            issue(b0 + 1, 1)
            wait(b0, 0)
            process(b0, 0)
            issue(b0 + 2, 0)
            wait(b0 + 1, 1)
            process(b0 + 1, 1)

        wait(NB - 1, 0)
        process(NB - 1, 0)

        plsc.subcore_barrier()

        @pl.when(sid < 10)
        def _(c=c):
            @pl.loop(0, 25)
            def _(k):
                pltpu.sync_copy(
                    acc_s.at[pl.ds(sid * 1000 + k * 40, 40)], tmp_v)
                pltpu.sync_copy(
                    tmp_v,
                    parts_o[c].at[cid, pl.ds(sid * 1000 + k * 40, 40)])


def _sc_layer(nc, src3, dst3, ae3, s, d, z128, z1d, hcs):
    alpha = pl.kernel(
        _sc_alpha_body,
        out_type=[jax.ShapeDtypeStruct((NT, NB, EB), jnp.float32),
                  jax.ShapeDtypeStruct((N,), jnp.float32),
                  jax.ShapeDtypeStruct((N,), jnp.float32)],
        mesh=_mesh(),
        scratch_types=[
            pltpu.VMEM((N,), jnp.float32),
            pltpu.VMEM((N,), jnp.float32),
            pltpu.VMEM((NB, EB), jnp.float32),
            pltpu.VMEM((NB, EB), jnp.int32),
            pltpu.VMEM((NB, EB), jnp.int32),
            pltpu.VMEM((NB, EB), jnp.float32),
            pltpu.VMEM((1000,), jnp.float32),
            pltpu.VMEM((1000,), jnp.float32),
            pltpu.VMEM_SHARED((N,), jnp.float32),
            pltpu.SemaphoreType.DMA,
        ],
        compiler_params=pltpu.CompilerParams(needs_layout_passes=False),
    )
    ex3, den0, den1 = alpha(src3, dst3, ae3, s, d, z1d)

    agg = pl.kernel(
        functools.partial(_sc_agg_body, nc),
        out_type=[jax.ShapeDtypeStruct((2, N, F), jnp.float32)] * nc,
        mesh=_mesh(),
        scratch_types=[
            pltpu.VMEM((NB, EB), jnp.int32),
            pltpu.VMEM((2, EB), jnp.int32),
            pltpu.VMEM((2, EB), jnp.int32),
            pltpu.VMEM((2, 1, EB), jnp.float32),
            pltpu.VMEM((EB, F), jnp.float32),
            pltpu.VMEM((EB, F), jnp.float32),
            pltpu.VMEM((40, F), jnp.float32),
            pltpu.VMEM_SHARED((N, F), jnp.float32),
            pltpu.SemaphoreType.DMA,
        ],
        compiler_params=pltpu.CompilerParams(needs_layout_passes=False),
    )
    ex4 = ex3.reshape(NT, NB, 1, EB)
    pk3 = jnp.left_shift(dst3, 16) | src3
    parts = agg(pk3, ex4, z128, *hcs)
    return jnp.stack([den0, den1]), list(parts)


def _post_body(nc, *refs):
    parts = refs[:nc]
    hrefs = refs[nc:2 * nc]
    sd_ref = refs[2 * nc]
    den_ref = refs[2 * nc + 1]
    cl_ref = refs[2 * nc + 2]
    bp_ref = refs[2 * nc + 3]
    yrefs = refs[2 * nc + 4:3 * nc + 4]
    strefs = refs[3 * nc + 4:]

    sd = sd_ref[...]
    cl = cl_ref[0:1, 0:1]
    al = sd[:, 0:1] + sd[:, 1:2] + cl
    al = jnp.maximum(al, al * 0.2)
    exl = jnp.exp(al)
    den = den_ref[...]
    dtot = den[0] + den[1] + exl + 1e-16

    i = pl.program_id(0)
    for c in range(nc):
        p = parts[c][...]
        h = hrefs[c][...]
        y = (p[0] + p[1] + exl * h) / dtot + bp_ref[c:c + 1, :]
        y = jnp.maximum(y, y * 0.01)
        yrefs[c][...] = y

        @pl.when(i == 0)
        def _():
            strefs[c][...] = jnp.zeros((8, F), jnp.float32)

        srow = jnp.sum(y, axis=0)
        sqrow = jnp.sum(y * y, axis=0)
        strefs[c][...] += jnp.concatenate(
            [srow[None, :], sqrow[None, :], jnp.zeros((6, F), jnp.float32)],
            axis=0)


def _post_layer(nc, parts, hcs, sdpack, denp3, clpack, bp8):
    blk = pl.BlockSpec((RB, F), lambda i: (i, 0))
    cblk = pl.BlockSpec((8, F), lambda i: (0, 0))
    in_specs = [pl.BlockSpec((2, RB, F), lambda i: (0, i, 0))] * nc
    in_specs += [blk] * nc
    in_specs += [blk,
                 pl.BlockSpec((2, RB, 1), lambda i: (0, i, 0)),
                 cblk, cblk]
    out_shape = ([jax.ShapeDtypeStruct((N, F), jnp.float32)] * nc
                 + [jax.ShapeDtypeStruct((8, F), jnp.float32)] * nc)
    res = pl.pallas_call(
        functools.partial(_post_body, nc),
        grid=(NRB,),
        in_specs=in_specs,
        out_specs=[blk] * nc + [cblk] * nc,
        out_shape=out_shape,
    )(*parts, *hcs, sdpack, denp3, clpack, bp8)
    return list(res[:nc]), list(res[nc:])


def _pool_body(ncin, *refs):
    zrefs = refs[:ncin]
    strefs = refs[ncin:2 * ncin]
    (g_ref, be_ref, b3_ref, fw0_ref, fw1_ref, fw2_ref, fb_ref,
     pool_ref, out_ref) = refs[2 * ncin:]

    i = pl.program_id(0)
    zs = []
    for cin in range(ncin):
        z = zrefs[cin][...]
        st = strefs[cin][...]
        m = st[0:1, :] / N
        var = st[1:2, :] / N - m * m
        inv = lax.rsqrt(var + 1e-5)
        zs.append((z - m) * inv * g_ref[cin:cin + 1, :]
                  + be_ref[cin:cin + 1, :])
    zfull = jnp.concatenate(zs, axis=1)
    b = b3_ref[0, 0, :]
    oh = (lax.broadcasted_iota(jnp.int32, (G, RB), 0)
          == b[None, :]).astype(jnp.float32)

    @pl.when(i == 0)
    def _():
        pool_ref[...] = jnp.zeros_like(pool_ref)

    pool_ref[...] += jnp.dot(oh, zfull, preferred_element_type=jnp.float32,
                             precision=lax.Precision.HIGHEST)

    @pl.when(i == NRB - 1)
    def _():
        p = pool_ref[...]
        z1 = jnp.dot(p, fw0_ref[...],
                     preferred_element_type=jnp.float32) + fb_ref[0:1, :]
        z1 = jnp.maximum(z1, 0.0)
        z2 = jnp.dot(z1, fw1_ref[...],
                     preferred_element_type=jnp.float32) + fb_ref[1:2, :16]
        z2 = jnp.maximum(z2, 0.0)
        z3 = jnp.dot(z2, fw2_ref[...],
                     preferred_element_type=jnp.float32) + fb_ref[2:3, :8]
        out_ref[...] = z3


def _pool_layer(ncin, zchunks, stats, g8, be8, batch3, fw0, fw1, fw2p, fbp):
    blk = pl.BlockSpec((RB, F), lambda i: (i, 0))
    cblk = pl.BlockSpec((8, F), lambda i: (0, 0))
    din = ncin * F
    in_specs = [blk] * ncin + [cblk] * ncin
    in_specs += [cblk, cblk,
                 pl.BlockSpec((1, 1, RB), lambda i: (i, 0, 0)),
                 pl.BlockSpec((din, F), lambda i: (0, 0)),
                 pl.BlockSpec((F, 16), lambda i: (0, 0)),
                 pl.BlockSpec((16, 8), lambda i: (0, 0))]
    in_specs += [cblk]
    out_shape = [jax.ShapeDtypeStruct((G, din), jnp.float32),
                 jax.ShapeDtypeStruct((G, 8), jnp.float32)]
    res = pl.pallas_call(
        functools.partial(_pool_body, ncin),
        grid=(NRB,),
        in_specs=in_specs,
        out_specs=[pl.BlockSpec((G, din), lambda i: (0, 0)),
                   pl.BlockSpec((G, 8), lambda i: (0, 0))],
        out_shape=out_shape,
    )(*zchunks, *stats, g8, be8, batch3, fw0, fw1, fw2p, fbp)
    return res[1]


def _pad8(a):
    rows, cols = a.shape
    return jnp.concatenate(
        [a, jnp.zeros((8 - rows, cols), jnp.float32)], axis=0)


def kernel(x, edge_index, edge_attr, batch, params):
    wes = [params['We%d' % l] for l in range(5)]
    aep = jnp.zeros((8, 512), jnp.float32)
    for l in range(5):
        aep = aep.at[l, :_DIMS[l][1]].set(params['ae%d' % l])
    w3s, aas, g8s, be8s, bp8s = [], [], [], [], []
    for l, (din, dout) in enumerate(_DIMS):
        W = params['W%d' % l]
        aa = jnp.zeros((8, dout), jnp.float32)
        aa = aa.at[0, :].set(params['as%d' % l])
        aa = aa.at[1, :].set(params['ad%d' % l])
        aas.append(aa)
        w3s.append(W.reshape(din // F, F, dout))
        g8s.append(_pad8(params['g%d' % l].reshape(dout // F, F)))
        be8s.append(_pad8(params['beta%d' % l].reshape(dout // F, F)))
        bp8s.append(_pad8(params['b%d' % l].reshape(dout // F, F)))
    fw0 = params['fcw0']
    fw1 = params['fcw1']
    fw2p = jnp.concatenate(
        [params['fcw2'], jnp.zeros((16, 7), jnp.float32)], axis=1)
    fbp = jnp.zeros((8, F), jnp.float32)
    fbp = fbp.at[0, :].set(params['fcb0'])
    fbp = fbp.at[1, :16].set(params['fcb1'])
    fbp = fbp.at[2, :1].set(params['fcb2'])

    ae_list, sum_ea = _alpha_e(edge_attr, wes, aep)
    mean_ea = sum_ea[0, :] / E
    cvals = jnp.stack([(mean_ea @ wes[l]) @ params['ae%d' % l]
                       for l in range(5)])

    src3 = edge_index[0].reshape(NT, NB, EB)
    dst3 = edge_index[1].reshape(NT, NB, EB)
    z128 = jnp.zeros((N, F), jnp.float32)
    z1d = jnp.zeros((N,), jnp.float32)
    batch3 = batch.reshape(NRB, 1, RB)

    zchunks = [x]
    stats = None
    for l, (din, dout) in enumerate(_DIMS):
        nc = dout // F
        hcs, sdpack = _matmul_layer(
            zchunks, stats, None if stats is None else g8s[l - 1],
            None if stats is None else be8s[l - 1],
            w3s[l], aas[l], nc)
        s = sdpack[:, 0]
        d = sdpack[:, 1]
        ae3 = ae_list[l].reshape(NT, NB, EB)
        denomp, parts = _sc_layer(
            nc, src3, dst3, ae3, s, d, z128, z1d, hcs)
        denp3 = denomp.reshape(2, N, 1)
        clpack = jnp.full((8, F), cvals[l], jnp.float32)
        zchunks, stats = _post_layer(
            nc, parts, hcs, sdpack, denp3, clpack, bp8s[l])

    out8 = _pool_layer(
        len(zchunks), zchunks, stats, g8s[4], be8s[4], batch3,
        fw0, fw1, fw2p, fbp)
    return out8[:, 0]
